# baseline (speedup 1.0000x reference)
import jax
import jax.numpy as jnp
from jax.experimental import pallas as pl
from jax.experimental.pallas import tpu as pltpu

_N = 4096
_TAU = 0.2
_BE = 512
_BM = 256
_BL = 1024
_F32 = jnp.float32
_BF16 = jnp.bfloat16


def _relu(v):
    return jnp.maximum(v, 0.0)


def _dotb(a, b):
    return jnp.dot(a.astype(_BF16), b.astype(_BF16),
                   preferred_element_type=_F32)


def _diff_body(lap_ref, x_ref, w1_ref, b1_ref, w2_ref, b2_ref,
               wg0_ref, bg0_ref, tg0_ref, wl0_ref, bl0_ref, tl0_ref,
               wg1_ref, bg1_ref, tg1_ref, wl1_ref, bl1_ref, tl1_ref,
               wp1_ref, bp1_ref, wp2_ref, bp2_ref,
               xg_ref, xl_ref, g_ref, l_ref,
               lapb_s, hb_s, xg1b_s, xl1b_s):
    s = pl.program_id(0)
    m = pl.program_id(1)
    rm = pl.ds(m * _BM, _BM)

    @pl.when(s == 0)
    def _phase_enc():
        lapb_s[rm, :] = lap_ref[...].astype(_BF16)
        h = _relu(jnp.dot(x_ref[...], w1_ref[...],
                          preferred_element_type=_F32) + b1_ref[...])
        h = jnp.dot(h, w2_ref[...], preferred_element_type=_F32) + b2_ref[...]
        hb_s[rm, :] = _relu(h).astype(_BF16)

    @pl.when(s == 1)
    def _phase_d1():
        a = lapb_s[rm, :]
        lx = jnp.dot(a, hb_s[...], preferred_element_type=_F32)
        hm = hb_s[rm, :].astype(_F32)
        ug = hm - tg0_ref[...] * lx
        ul = hm - tl0_ref[...] * lx
        xg = _relu(jnp.dot(ug, wg0_ref[...], preferred_element_type=_F32)
                   + bg0_ref[...])
        xl = _relu(jnp.dot(ul, wl0_ref[...], preferred_element_type=_F32)
                   + bl0_ref[...])
        xg1b_s[rm, :] = xg.astype(_BF16)
        xl1b_s[rm, :] = xl.astype(_BF16)

    @pl.when(s == 2)
    def _phase_d2():
        a = lapb_s[rm, :]
        lxg = jnp.dot(a, xg1b_s[...], preferred_element_type=_F32)
        lxl = jnp.dot(a, xl1b_s[...], preferred_element_type=_F32)
        ug = xg1b_s[rm, :].astype(_F32) - tg1_ref[...] * lxg
        ul = xl1b_s[rm, :].astype(_F32) - tl1_ref[...] * lxl
        xg = _relu(jnp.dot(ug, wg1_ref[...], preferred_element_type=_F32)
                   + bg1_ref[...])
        xl = _relu(jnp.dot(ul, wl1_ref[...], preferred_element_type=_F32)
                   + bl1_ref[...])
        xg_ref[...] = xg
        xl_ref[...] = xl
        zg = (jnp.dot(xg, wp1_ref[...], preferred_element_type=_F32)
              + bp1_ref[...])
        zl = (jnp.dot(xl, wp2_ref[...], preferred_element_type=_F32)
              + bp2_ref[...])
        ng = jnp.sqrt(jnp.sum(zg * zg, axis=1, keepdims=True))
        nl = jnp.sqrt(jnp.sum(zl * zl, axis=1, keepdims=True))
        g_ref[...] = (zg / jnp.maximum(ng, 1e-12)).astype(_BF16)
        l_ref[...] = (zl / jnp.maximum(nl, 1e-12)).astype(_BF16)


_LW = 128


def _lane_part(t):
    acc = t[:, 0:_LW]
    for k in range(_LW, _BL, _LW):
        acc = acc + t[:, k:k + _LW]
    return acc


def _sub_part(t):
    acc = t[0:_LW, :]
    for k in range(_LW, _BL, _LW):
        acc = acc + t[k:k + _LW, :]
    return acc


_K2 = 1.4426950408889634 / _TAU


def _loss_body(g_ref, l_ref, loss_ref, rc_p, cc_p, ra_lp, ra_sp, rb_lp,
               rb_sp, eye_s):
    i = pl.program_id(0)
    j = pl.program_id(1)
    nb = _N // _BL
    dn = (((1,), (1,)), ((), ()))

    @pl.when(jnp.logical_and(i == 0, j == 0))
    def _init():
        rc_p[...] = jnp.zeros_like(rc_p)
        cc_p[...] = jnp.zeros_like(cc_p)
        ra_lp[...] = jnp.zeros_like(ra_lp)
        ra_sp[...] = jnp.zeros_like(ra_sp)
        rb_lp[...] = jnp.zeros_like(rb_lp)
        rb_sp[...] = jnp.zeros_like(rb_sp)
        rr = jax.lax.broadcasted_iota(jnp.int32, (_BL, _BL), 0)
        cc = jax.lax.broadcasted_iota(jnp.int32, (_BL, _BL), 1)
        eye_s[...] = (rr == cc).astype(_F32)

    gi = g_ref[pl.ds(i * _BL, _BL), :]
    gj = g_ref[pl.ds(j * _BL, _BL), :]
    li = l_ref[pl.ds(i * _BL, _BL), :]
    lj = l_ref[pl.ds(j * _BL, _BL), :]
    ec = jnp.exp2(
        jax.lax.dot_general(gi, lj, dn, preferred_element_type=_F32) * _K2)
    ri = pl.ds(i * _BL, _BL)
    sj = pl.ds(j * _LW, _LW)

    @pl.when(i == j)
    def _diag_tile():
        neg = 1.0 - eye_s[...]
        ecm = ec * neg
        rc_p[ri, :] += _lane_part(ecm)
        cc_p[sj, :] += _sub_part(ecm)
        ea = jnp.exp2(
            jax.lax.dot_general(gi, gj, dn,
                                preferred_element_type=_F32) * _K2) * neg
        ra_lp[ri, :] += _lane_part(ea)
        eb = jnp.exp2(
            jax.lax.dot_general(li, lj, dn,
                                preferred_element_type=_F32) * _K2) * neg
        rb_lp[ri, :] += _lane_part(eb)

    @pl.when(j > i)
    def _upper_tile():
        rc_p[ri, :] += _lane_part(ec)
        cc_p[sj, :] += _sub_part(ec)
        ea = jnp.exp2(
            jax.lax.dot_general(gi, gj, dn,
                                preferred_element_type=_F32) * _K2)
        ra_lp[ri, :] += _lane_part(ea)
        ra_sp[sj, :] += _sub_part(ea)
        eb = jnp.exp2(
            jax.lax.dot_general(li, lj, dn,
                                preferred_element_type=_F32) * _K2)
        rb_lp[ri, :] += _lane_part(eb)
        rb_sp[sj, :] += _sub_part(eb)

    @pl.when(j < i)
    def _lower_tile():
        rc_p[ri, :] += _lane_part(ec)
        cc_p[sj, :] += _sub_part(ec)

    @pl.when(jnp.logical_and(i == nb - 1, j == nb - 1))
    def _finalize():
        tot = 0.0
        for b in range(nb):
            rb_rows = pl.ds(b * _BL, _BL)
            sb_rows = pl.ds(b * _LW, _LW)
            ra = (jnp.sum(ra_lp[rb_rows, :], axis=1)
                  + jnp.sum(ra_sp[sb_rows, :], axis=0))
            rbv = (jnp.sum(rb_lp[rb_rows, :], axis=1)
                   + jnp.sum(rb_sp[sb_rows, :], axis=0))
            rc = jnp.sum(rc_p[rb_rows, :], axis=1)
            ccv = jnp.sum(cc_p[sb_rows, :], axis=0)
            tot += (jnp.sum(jnp.log(ra)) + jnp.sum(jnp.log(rbv))
                    + jnp.sum(jnp.log(rc)) + jnp.sum(jnp.log(ccv)))
        dg = jnp.sum(g_ref[...].astype(_F32) * l_ref[...].astype(_F32)) / _TAU
        loss_ref[...] = jnp.reshape(-0.5 * (2.0 * dg - tot) / _N, (1, 1))


def kernel(Lap, x, W1, b1, W2, b2, tg, Wg, bg, tl, Wl, bl, Wp1, bp1, Wp2,
           bp2):
    n, d_in = x.shape
    h1 = W1.shape[0]
    h2 = W2.shape[0]

    full = lambda shape: pl.BlockSpec(shape, lambda *_: (0,) * len(shape))

    tgv = [jnp.broadcast_to(tg[i], (1, h2)).astype(_F32) for i in range(2)]
    tlv = [tl[i].reshape(1, h2) for i in range(2)]

    first = lambda s, m: (jnp.where(s == 0, m, 0), 0)
    xg2, xl2, g, l = pl.pallas_call(
        _diff_body,
        grid=(3, n // _BM),
        in_specs=[
            pl.BlockSpec((_BM, n), first),
            pl.BlockSpec((_BM, d_in), first),
        ] + [pl.BlockSpec(sh, lambda s, m: (0, 0)) for sh in
             [(d_in, h1), (1, h1), (h1, h2), (1, h2)]
             + [(h2, h2), (1, h2), (1, h2), (h2, h2), (1, h2), (1, h2)] * 2
             + [(h2, h2), (1, h2), (h2, h2), (1, h2)]],
        out_specs=[pl.BlockSpec((_BM, h2), lambda s, m: (m, 0))] * 4,
        out_shape=[jax.ShapeDtypeStruct((n, h2), _F32)] * 2
        + [jax.ShapeDtypeStruct((n, h2), _BF16)] * 2,
        scratch_shapes=[
            pltpu.VMEM((n, n), _BF16),
            pltpu.VMEM((n, h2), _BF16),
            pltpu.VMEM((n, h2), _BF16),
            pltpu.VMEM((n, h2), _BF16),
        ],
    )(Lap, x,
      W1.T, b1.reshape(1, h1), W2.T, b2.reshape(1, h2),
      Wg[0].T, bg[0].reshape(1, h2), tgv[0],
      Wl[0].T, bl[0].reshape(1, h2), tlv[0],
      Wg[1].T, bg[1].reshape(1, h2), tgv[1],
      Wl[1].T, bl[1].reshape(1, h2), tlv[1],
      Wp1.T, bp1.reshape(1, h2), Wp2.T, bp2.reshape(1, h2))

    nb = n // _BL
    loss2d = pl.pallas_call(
        _loss_body,
        grid=(nb, nb),
        in_specs=[full((n, h2)), full((n, h2))],
        out_specs=pl.BlockSpec((1, 1), lambda i, j: (0, 0)),
        out_shape=jax.ShapeDtypeStruct((1, 1), _F32),
        scratch_shapes=[
            pltpu.VMEM((n, _LW), _F32),
            pltpu.VMEM((nb * _LW, _BL), _F32),
            pltpu.VMEM((n, _LW), _F32),
            pltpu.VMEM((nb * _LW, _BL), _F32),
            pltpu.VMEM((n, _LW), _F32),
            pltpu.VMEM((nb * _LW, _BL), _F32),
            pltpu.VMEM((_BL, _BL), _F32),
        ],
    )(g, l)

    return (xg2, xl2, loss2d[0, 0])


# speedup vs baseline: 1.1178x; 1.1178x over previous
import jax
import jax.numpy as jnp
from jax.experimental import pallas as pl
from jax.experimental.pallas import tpu as pltpu

_N = 4096
_TAU = 0.2
_BE = 512
_BM = 512
_BL = 1024
_LW = 128
_F32 = jnp.float32
_BF16 = jnp.bfloat16
_K2 = 1.4426950408889634 / _TAU


def _relu(v):
    return jnp.maximum(v, 0.0)


def _enc_body(x_ref, w1_ref, b1_ref, w2_ref, b2_ref, h_ref, hb_ref):
    h = _relu(jnp.dot(x_ref[...], w1_ref[...], preferred_element_type=_F32)
              + b1_ref[...])
    h = jnp.dot(h, w2_ref[...], preferred_element_type=_F32) + b2_ref[...]
    h = _relu(h)
    h_ref[...] = h
    hb_ref[...] = h.astype(_BF16)


def _diff_body(lap_ref, h_ref, hb_ref, wg0_ref, bg0_ref, tg0_ref, wl0_ref,
               bl0_ref, tl0_ref, wg1_ref, bg1_ref, tg1_ref, wl1_ref, bl1_ref,
               tl1_ref, wp1_ref, bp1_ref, wp2_ref, bp2_ref,
               xg_ref, xl_ref, g_ref, l_ref,
               lapb_s, xg1_s, xl1_s, xg1b_s, xl1b_s):
    s = pl.program_id(0)
    m = pl.program_id(1)
    rm = pl.ds(m * _BM, _BM)

    @pl.when(s == 0)
    def _step0():
        a = lap_ref[...].astype(_BF16)
        lapb_s[rm, :] = a
        lx = jnp.dot(a, hb_ref[...], preferred_element_type=_F32)
        hm = h_ref[...]
        ug = hm - tg0_ref[...] * lx
        ul = hm - tl0_ref[...] * lx
        xg = _relu(jnp.dot(ug, wg0_ref[...], preferred_element_type=_F32)
                   + bg0_ref[...])
        xl = _relu(jnp.dot(ul, wl0_ref[...], preferred_element_type=_F32)
                   + bl0_ref[...])
        xg1_s[rm, :] = xg
        xl1_s[rm, :] = xl
        xg1b_s[rm, :] = xg.astype(_BF16)
        xl1b_s[rm, :] = xl.astype(_BF16)

    @pl.when(s == 1)
    def _step1():
        a = lapb_s[rm, :]
        lxg = jnp.dot(a, xg1b_s[...], preferred_element_type=_F32)
        lxl = jnp.dot(a, xl1b_s[...], preferred_element_type=_F32)
        ug = xg1_s[rm, :] - tg1_ref[...] * lxg
        ul = xl1_s[rm, :] - tl1_ref[...] * lxl
        xg = _relu(jnp.dot(ug, wg1_ref[...], preferred_element_type=_F32)
                   + bg1_ref[...])
        xl = _relu(jnp.dot(ul, wl1_ref[...], preferred_element_type=_F32)
                   + bl1_ref[...])
        xg_ref[...] = xg
        xl_ref[...] = xl
        zg = (jnp.dot(xg, wp1_ref[...], preferred_element_type=_F32)
              + bp1_ref[...])
        zl = (jnp.dot(xl, wp2_ref[...], preferred_element_type=_F32)
              + bp2_ref[...])
        ng = jnp.sqrt(jnp.sum(zg * zg, axis=1, keepdims=True))
        nl = jnp.sqrt(jnp.sum(zl * zl, axis=1, keepdims=True))
        g_ref[...] = (zg / jnp.maximum(ng, 1e-12)).astype(_BF16)
        l_ref[...] = (zl / jnp.maximum(nl, 1e-12)).astype(_BF16)


def _lane_part(t):
    acc = t[:, 0:_LW]
    for k in range(_LW, _BL, _LW):
        acc = acc + t[:, k:k + _LW]
    return acc


def _sub_part(t):
    acc = t[0:_LW, :]
    for k in range(_LW, _BL, _LW):
        acc = acc + t[k:k + _LW, :]
    return acc


def _loss_body(g_ref, l_ref, loss_ref, rc_p, cc_p, ra_lp, ra_sp, rb_lp,
               rb_sp, eye_s):
    i = pl.program_id(0)
    j = pl.program_id(1)
    nb = _N // _BL
    dn = (((1,), (1,)), ((), ()))

    @pl.when(jnp.logical_and(i == 0, j == 0))
    def _init():
        rc_p[...] = jnp.zeros_like(rc_p)
        cc_p[...] = jnp.zeros_like(cc_p)
        ra_lp[...] = jnp.zeros_like(ra_lp)
        ra_sp[...] = jnp.zeros_like(ra_sp)
        rb_lp[...] = jnp.zeros_like(rb_lp)
        rb_sp[...] = jnp.zeros_like(rb_sp)
        rr = jax.lax.broadcasted_iota(jnp.int32, (_BL, _BL), 0)
        cc = jax.lax.broadcasted_iota(jnp.int32, (_BL, _BL), 1)
        eye_s[...] = (rr == cc).astype(_F32)

    gi = g_ref[pl.ds(i * _BL, _BL), :]
    gj = g_ref[pl.ds(j * _BL, _BL), :]
    li = l_ref[pl.ds(i * _BL, _BL), :]
    lj = l_ref[pl.ds(j * _BL, _BL), :]
    ec = jnp.exp2(
        jax.lax.dot_general(gi, lj, dn, preferred_element_type=_F32) * _K2)
    ri = pl.ds(i * _BL, _BL)
    sj = pl.ds(j * _LW, _LW)

    @pl.when(i == j)
    def _diag_tile():
        neg = 1.0 - eye_s[...]
        ecm = ec * neg
        rc_p[ri, :] += _lane_part(ecm)
        cc_p[sj, :] += _sub_part(ecm)
        ea = jnp.exp2(
            jax.lax.dot_general(gi, gj, dn,
                                preferred_element_type=_F32) * _K2) * neg
        ra_lp[ri, :] += _lane_part(ea)
        eb = jnp.exp2(
            jax.lax.dot_general(li, lj, dn,
                                preferred_element_type=_F32) * _K2) * neg
        rb_lp[ri, :] += _lane_part(eb)

    @pl.when(j > i)
    def _upper_tile():
        rc_p[ri, :] += _lane_part(ec)
        cc_p[sj, :] += _sub_part(ec)
        ea = jnp.exp2(
            jax.lax.dot_general(gi, gj, dn,
                                preferred_element_type=_F32) * _K2)
        ra_lp[ri, :] += _lane_part(ea)
        ra_sp[sj, :] += _sub_part(ea)
        eb = jnp.exp2(
            jax.lax.dot_general(li, lj, dn,
                                preferred_element_type=_F32) * _K2)
        rb_lp[ri, :] += _lane_part(eb)
        rb_sp[sj, :] += _sub_part(eb)

    @pl.when(j < i)
    def _lower_tile():
        rc_p[ri, :] += _lane_part(ec)
        cc_p[sj, :] += _sub_part(ec)

    @pl.when(jnp.logical_and(i == nb - 1, j == nb - 1))
    def _finalize():
        tot = 0.0
        for b in range(nb):
            rb_rows = pl.ds(b * _BL, _BL)
            sb_rows = pl.ds(b * _LW, _LW)
            ra = (jnp.sum(ra_lp[rb_rows, :], axis=1)
                  + jnp.sum(ra_sp[sb_rows, :], axis=0))
            rbv = (jnp.sum(rb_lp[rb_rows, :], axis=1)
                   + jnp.sum(rb_sp[sb_rows, :], axis=0))
            rc = jnp.sum(rc_p[rb_rows, :], axis=1)
            ccv = jnp.sum(cc_p[sb_rows, :], axis=0)
            tot += (jnp.sum(jnp.log(ra)) + jnp.sum(jnp.log(rbv))
                    + jnp.sum(jnp.log(rc)) + jnp.sum(jnp.log(ccv)))
        dg = jnp.sum(g_ref[...].astype(_F32) * l_ref[...].astype(_F32)) / _TAU
        loss_ref[...] = jnp.reshape(-0.5 * (2.0 * dg - tot) / _N, (1, 1))


def kernel(Lap, x, W1, b1, W2, b2, tg, Wg, bg, tl, Wl, bl, Wp1, bp1, Wp2,
           bp2):
    n, d_in = x.shape
    h1 = W1.shape[0]
    h2 = W2.shape[0]

    full = lambda shape: pl.BlockSpec(shape, lambda *_: (0,) * len(shape))

    h, hb = pl.pallas_call(
        _enc_body,
        grid=(n // _BE,),
        in_specs=[
            pl.BlockSpec((_BE, d_in), lambda m: (m, 0)),
            full((d_in, h1)),
            full((1, h1)),
            full((h1, h2)),
            full((1, h2)),
        ],
        out_specs=[pl.BlockSpec((_BE, h2), lambda m: (m, 0))] * 2,
        out_shape=[jax.ShapeDtypeStruct((n, h2), _F32),
                   jax.ShapeDtypeStruct((n, h2), _BF16)],
    )(x, W1.T, b1.reshape(1, h1), W2.T, b2.reshape(1, h2))

    tgv = [jnp.broadcast_to(tg[i], (1, h2)).astype(_F32) for i in range(2)]
    tlv = [tl[i].reshape(1, h2) for i in range(2)]

    xg2, xl2, g, l = pl.pallas_call(
        _diff_body,
        grid=(2, n // _BM),
        in_specs=[
            pl.BlockSpec((_BM, n), lambda s, m: ((1 - s) * m, 0)),
            pl.BlockSpec((_BM, h2), lambda s, m: (m, 0)),
            pl.BlockSpec((n, h2), lambda s, m: (0, 0)),
        ] + [pl.BlockSpec(sh, lambda s, m: (0, 0)) for sh in
             [(h2, h2), (1, h2), (1, h2), (h2, h2), (1, h2), (1, h2)] * 2
             + [(h2, h2), (1, h2), (h2, h2), (1, h2)]],
        out_specs=[pl.BlockSpec((_BM, h2), lambda s, m: (m, 0))] * 4,
        out_shape=[jax.ShapeDtypeStruct((n, h2), _F32)] * 2
        + [jax.ShapeDtypeStruct((n, h2), _BF16)] * 2,
        scratch_shapes=[
            pltpu.VMEM((n, n), _BF16),
            pltpu.VMEM((n, h2), _F32),
            pltpu.VMEM((n, h2), _F32),
            pltpu.VMEM((n, h2), _BF16),
            pltpu.VMEM((n, h2), _BF16),
        ],
    )(Lap, h, hb,
      Wg[0].T, bg[0].reshape(1, h2), tgv[0],
      Wl[0].T, bl[0].reshape(1, h2), tlv[0],
      Wg[1].T, bg[1].reshape(1, h2), tgv[1],
      Wl[1].T, bl[1].reshape(1, h2), tlv[1],
      Wp1.T, bp1.reshape(1, h2), Wp2.T, bp2.reshape(1, h2))

    nb = n // _BL
    loss2d = pl.pallas_call(
        _loss_body,
        grid=(nb, nb),
        in_specs=[full((n, h2)), full((n, h2))],
        out_specs=pl.BlockSpec((1, 1), lambda i, j: (0, 0)),
        out_shape=jax.ShapeDtypeStruct((1, 1), _F32),
        scratch_shapes=[
            pltpu.VMEM((n, _LW), _F32),
            pltpu.VMEM((nb * _LW, _BL), _F32),
            pltpu.VMEM((n, _LW), _F32),
            pltpu.VMEM((nb * _LW, _BL), _F32),
            pltpu.VMEM((n, _LW), _F32),
            pltpu.VMEM((nb * _LW, _BL), _F32),
            pltpu.VMEM((_BL, _BL), _F32),
        ],
    )(g, l)

    return (xg2, xl2, loss2d[0, 0])
